# baseline (speedup 1.0000x reference)
import functools

import jax
import jax.numpy as jnp
from jax import lax
from jax.experimental import pallas as pl
from jax.experimental.pallas import tpu as pltpu
from jax.experimental.pallas import tpu_sc as plsc

B = 4096
K = 512
L = 16
NC, NS = 2, 16
NW = NC * NS
JPT = B // NS
HJ = JPT // 2
RPT = B // NW


def _sc_body(t_hbm, p_hbm, s_hbm, st_hbm, se_hbm, hist_hbm,
             tj, pj, ej, bj, hist, hall, opb, offp, rank,
             posA, posB, ti, st, se, sufE, sv):
    c = lax.axis_index("c")
    s = lax.axis_index("s")
    w = s * NC + c

    jbase = s * JPT
    pltpu.sync_copy(t_hbm.at[pl.ds(jbase, JPT)], tj)
    pltpu.sync_copy(p_hbm.at[pl.ds(jbase, JPT)], pj)
    for q in range(K // L):
        hist[pl.ds(q * L, L)] = jnp.zeros((L,), jnp.int32)
        rank[pl.ds(q * L, L)] = jnp.zeros((L,), jnp.int32)
    cbase = None
    for q in range(JPT // L):
        tv = tj[pl.ds(q * L, L)]
        ej[pl.ds(q * L, L)] = jnp.exp(pj[pl.ds(q * L, L)])
        bv = jnp.clip((tv * jnp.float32(K)).astype(jnp.int32), 0, K - 1)
        bj[pl.ds(q * L, L)] = bv
        cnt, last = plsc.scan_count(bv)
        if cbase is None:
            cbase = jnp.min(cnt)
        plsc.addupdate_scatter(hist, [bv], cnt - cbase + 1, mask=last)

    pltpu.sync_copy(hist, hist_hbm.at[s])
    plsc.subcore_barrier()
    pltpu.sync_copy(hist_hbm, hall)

    carry = jnp.int32(0)
    for q in range(K // L):
        tot = jnp.zeros((L,), jnp.int32)
        base = jnp.zeros((L,), jnp.int32)
        for s2 in range(NS):
            v = hall[s2, pl.ds(q * L, L)]
            tot = tot + v
            before = jnp.full((L,), s2, jnp.int32) < s
            base = base + jnp.where(before, v, jnp.zeros_like(v))
        inc = plsc.cumsum(tot)
        off_chunk = inc - tot + carry
        offp[pl.ds(q * L, L)] = off_chunk
        opb[pl.ds(q * L, L)] = off_chunk + base
        carry = carry + jnp.sum(tot)
    offp[pl.ds(K, L)] = jnp.full((L,), B, jnp.int32)

    for q in range(JPT // L):
        bv = bj[pl.ds(q * L, L)]
        old = plsc.load_gather(rank, [bv])
        cnt, last = plsc.scan_count(bv)
        r0 = cnt - cbase
        pos = plsc.load_gather(opb, [bv]) + old + r0
        if q < HJ // L:
            posA[pl.ds(q * L, L)] = pos
        else:
            posB[pl.ds(q * L - HJ, L)] = pos
        plsc.addupdate_scatter(rank, [bv], r0 + 1, mask=last)

    pltpu.sync_copy(tj.at[pl.ds(0, HJ)], st_hbm.at[posA])
    pltpu.sync_copy(tj.at[pl.ds(HJ, HJ)], st_hbm.at[posB])
    pltpu.sync_copy(ej.at[pl.ds(0, HJ)], se_hbm.at[posA])
    pltpu.sync_copy(ej.at[pl.ds(HJ, HJ)], se_hbm.at[posB])
    plsc.subcore_barrier()

    pltpu.sync_copy(st_hbm, st)
    pltpu.sync_copy(se_hbm, se)

    sufE[pl.ds(B, L)] = jnp.zeros((L,), jnp.float32)

    def _suf_body(q, carryf):
        q2 = (B // L - 1) - q
        v = se[pl.ds(q2 * L, L)]
        rc = plsc.cumsum(lax.rev(v, (0,)))
        sufE[pl.ds(q2 * L, L)] = lax.rev(rc, (0,)) + carryf
        return carryf + jnp.sum(v)

    lax.fori_loop(0, B // L, _suf_body, jnp.float32(0.0))

    rbase = w * RPT
    pltpu.sync_copy(t_hbm.at[pl.ds(rbase, RPT)], ti)
    for g in range(RPT // L):
        tv = ti[pl.ds(g * L, L)]
        bv = jnp.clip((tv * jnp.float32(K)).astype(jnp.int32), 0, K - 1)
        begin = plsc.load_gather(offp, [bv])
        end = plsc.load_gather(offp, [bv + 1])
        acc = plsc.load_gather(sufE, [end])
        maxm = jnp.max(end - begin)

        def _wcond(state):
            s2, _ = state
            return s2 < maxm

        def _wbody(state):
            s2, a = state
            idx = begin + s2
            inb = idx < end
            idxc = jnp.minimum(idx, B - 1)
            stv = plsc.load_gather(st, [idxc])
            sev = plsc.load_gather(se, [idxc])
            take = jnp.logical_and(inb, stv >= tv)
            a = a + jnp.where(take, sev, jnp.zeros_like(sev))
            return s2 + 1, a

        _, acc = lax.while_loop(_wcond, _wbody, (jnp.int32(0), acc))
        sv[pl.ds(g * L, L)] = acc
    pltpu.sync_copy(sv, s_hbm.at[pl.ds(rbase, RPT)])


def _make_sc_call(interpret=False):
    mesh = plsc.VectorSubcoreMesh(
        core_axis_name="c", subcore_axis_name="s",
        num_cores=NC, num_subcores=NS)
    return pl.kernel(
        _sc_body,
        out_type=(
            jax.ShapeDtypeStruct((B,), jnp.float32),
            jax.ShapeDtypeStruct((B,), jnp.float32),
            jax.ShapeDtypeStruct((B,), jnp.float32),
            jax.ShapeDtypeStruct((NS, K), jnp.int32),
        ),
        mesh=mesh,
        scratch_types=[
            pltpu.VMEM((JPT,), jnp.float32),
            pltpu.VMEM((JPT,), jnp.float32),
            pltpu.VMEM((JPT,), jnp.float32),
            pltpu.VMEM((JPT,), jnp.int32),
            pltpu.VMEM((K,), jnp.int32),
            pltpu.VMEM((NS, K), jnp.int32),
            pltpu.VMEM((K,), jnp.int32),
            pltpu.VMEM((K + L,), jnp.int32),
            pltpu.VMEM((K,), jnp.int32),
            pltpu.VMEM((HJ,), jnp.int32),
            pltpu.VMEM((HJ,), jnp.int32),
            pltpu.VMEM((RPT,), jnp.float32),
            pltpu.VMEM((B,), jnp.float32),
            pltpu.VMEM((B,), jnp.float32),
            pltpu.VMEM((B + L,), jnp.float32),
            pltpu.VMEM((RPT,), jnp.float32),
        ],
        compiler_params=pltpu.CompilerParams(needs_layout_passes=False),
        interpret=interpret,
    )


def _fin_body(p_ref, ind_ref, s_ref, out_ref):
    lgs = jnp.log(s_ref[...])
    ind = ind_ref[...]
    num = jnp.sum(ind * (p_ref[...] - lgs))
    den = jnp.sum(ind)
    out_ref[...] = (-(num / den)).reshape(1, 1)


@jax.jit
def kernel(pred, gt_indicator, gt_time):
    p = pred.reshape(B)
    sc = _make_sc_call()
    s_arr, _, _, _ = sc(gt_time, p)

    p2 = p.reshape(32, 128)
    ind2 = gt_indicator.astype(jnp.float32).reshape(32, 128)
    s2 = s_arr.reshape(32, 128)
    out = pl.pallas_call(
        _fin_body,
        out_shape=jax.ShapeDtypeStruct((1, 1), jnp.float32),
    )(p2, ind2, s2)
    return out[0, 0]
